# baseline (speedup 1.0000x reference)
import functools

import jax
import jax.numpy as jnp
from jax import lax
from jax.experimental import pallas as pl
from jax.experimental.pallas import tpu as pltpu
from jax.experimental.pallas import tpu_sc as plsc

L = 16


def _entity_pool_sc(table, idx, masks):
    n_rows, D = table.shape
    BE, K = idx.shape
    info = plsc.get_sparse_core_info()
    nw = info.num_cores * info.num_subcores
    epw = BE // nw

    mesh = plsc.VectorSubcoreMesh(core_axis_name="c", subcore_axis_name="s")

    @functools.partial(
        pl.kernel,
        mesh=mesh,
        out_type=jax.ShapeDtypeStruct((BE, D), jnp.float32),
        scratch_types=[
            pltpu.VMEM((epw, K), jnp.int32),
            pltpu.VMEM((epw, K), jnp.int32),
            pltpu.VMEM((K, D), jnp.float32),
            pltpu.VMEM((K, D), jnp.float32),
            pltpu.VMEM((epw, D), jnp.float32),
            pltpu.SemaphoreType.DMA,
            pltpu.SemaphoreType.DMA,
        ],
    )
    def run(table_hbm, idx_hbm, mask_hbm, out_hbm,
            idx_v, mask_v, buf0, buf1, out_v, sem0, sem1):
        wid = lax.axis_index("s") * info.num_cores + lax.axis_index("c")
        base = wid * epw
        pltpu.sync_copy(idx_hbm.at[pl.ds(base, epw), :], idx_v)
        pltpu.sync_copy(mask_hbm.at[pl.ds(base, epw), :], mask_v)

        bufs = (buf0, buf1)
        sems = (sem0, sem1)

        def start(e):
            pltpu.make_async_copy(
                table_hbm.at[idx_v.at[e]], bufs[e % 2], sems[e % 2]
            ).start()

        def wait(e):
            pltpu.make_async_copy(
                table_hbm.at[idx_v.at[e]], bufs[e % 2], sems[e % 2]
            ).wait()

        start(0)
        start(1)
        for e in range(epw):
            wait(e)
            buf = bufs[e % 2]
            splats = []
            for h in range(K // L):
                mv = mask_v[e, pl.ds(h * L, L)]
                bv = jnp.where(mv == 0, jnp.float32(-1e30), jnp.float32(0.0))
                for j in range(L):
                    splats.append(jnp.full((L,), bv[j], dtype=jnp.float32))

            def cbody(c, carry, buf=buf, splats=splats, e=e):
                off = c * L
                acc = buf[0, pl.ds(off, L)] + splats[0]
                for kk in range(1, K):
                    acc = jnp.maximum(acc, buf[kk, pl.ds(off, L)] + splats[kk])
                out_v[e, pl.ds(off, L)] = acc
                return carry

            lax.fori_loop(0, D // L, cbody, 0)
            if e + 2 < epw:
                start(e + 2)
        pltpu.sync_copy(out_v, out_hbm.at[pl.ds(base, epw), :])

    return run(table, idx, masks)


def _entity_pool_tc(table4, ents, masks):
    Bt, E, K = ents.shape
    M = table4.shape[1]
    ents_flat = ents.reshape(-1)
    masks_flat = masks.reshape(-1)

    def body(ents_s, masks_s, table_ref, out_ref):
        b = pl.program_id(0)
        e = pl.program_id(1)
        base = (b * E + e) * K
        acc = None
        for k in range(K):
            idx = ents_s[base + k]
            m = masks_s[base + k]
            bias = jnp.where(m == 0, jnp.float32(-1e30), jnp.float32(0.0))
            row = table_ref[0, idx] + bias
            acc = row if acc is None else jnp.maximum(acc, row)
        out_ref[0, 0] = acc

    grid_spec = pltpu.PrefetchScalarGridSpec(
        num_scalar_prefetch=2,
        grid=(Bt, E),
        in_specs=[
            pl.BlockSpec((1, M, 8, 128), lambda b, e, *_: (b, 0, 0, 0)),
        ],
        out_specs=pl.BlockSpec((1, 1, 8, 128), lambda b, e, *_: (b, e, 0, 0)),
    )
    return pl.pallas_call(
        body,
        grid_spec=grid_spec,
        out_shape=jax.ShapeDtypeStruct((Bt, E, 8, 128), jnp.float32),
    )(ents_flat, masks_flat, table4)


_TC_BATCHES = 4


def kernel(mention_reprs, entities, entity_masks):
    B, M, D = mention_reprs.shape
    _, E, K = entities.shape
    bt = _TC_BATCHES
    bs = B - bt

    table = mention_reprs[:bs].reshape(bs * M, D)
    idx = (entities[:bs] + (jnp.arange(bs, dtype=jnp.int32) * M)[:, None, None]
           ).reshape(bs * E, K)
    masks = entity_masks[:bs].reshape(bs * E, K)
    out_sc = _entity_pool_sc(table, idx, masks).reshape(bs, E, D)

    table4 = mention_reprs[bs:].reshape(bt, M, 8, 128)
    out_tc = _entity_pool_tc(table4, entities[bs:], entity_masks[bs:])
    out_tc = out_tc.reshape(bt, E, D)

    return jnp.concatenate([out_sc, out_tc], axis=0)


# speedup vs baseline: 2.5291x; 2.5291x over previous
import functools

import jax
import jax.numpy as jnp
from jax import lax
from jax.experimental import pallas as pl
from jax.experimental.pallas import tpu as pltpu
from jax.experimental.pallas import tpu_sc as plsc

L = 16


def _entity_pool_sc(table, idx, masks):
    n_rows, D = table.shape
    BE, K = idx.shape
    KH = K // 2
    info = plsc.get_sparse_core_info()
    nw = info.num_cores * info.num_subcores
    epw = BE // nw

    mesh = plsc.VectorSubcoreMesh(core_axis_name="c", subcore_axis_name="s")

    @functools.partial(
        pl.kernel,
        mesh=mesh,
        out_type=jax.ShapeDtypeStruct((BE, D), jnp.float32),
        scratch_types=[
            pltpu.VMEM((epw, K), jnp.int32),
            pltpu.VMEM((epw, K), jnp.int32),
            pltpu.VMEM((KH, D), jnp.float32),
            pltpu.VMEM((KH, D), jnp.float32),
            pltpu.VMEM((KH, D), jnp.float32),
            pltpu.VMEM((KH, D), jnp.float32),
            pltpu.VMEM((epw, D), jnp.float32),
            pltpu.SemaphoreType.DMA,
            pltpu.SemaphoreType.DMA,
            pltpu.SemaphoreType.DMA,
            pltpu.SemaphoreType.DMA,
        ],
    )
    def run(table_hbm, idx_hbm, mask_hbm, out_hbm,
            idx_v, mask_v, buf0, buf1, buf2, buf3, out_v,
            sem0, sem1, sem2, sem3):
        wid = lax.axis_index("s") * info.num_cores + lax.axis_index("c")
        base = wid * epw
        pltpu.sync_copy(idx_hbm.at[pl.ds(base, epw), :], idx_v)
        pltpu.sync_copy(mask_hbm.at[pl.ds(base, epw), :], mask_v)

        bufs = (buf0, buf1, buf2, buf3)
        sems = (sem0, sem1, sem2, sem3)

        def slot(e, h):
            return (2 * e + h) % 4

        def copy(e, h):
            s = slot(e, h)
            return pltpu.make_async_copy(
                table_hbm.at[idx_v.at[e, pl.ds(h * KH, KH)]], bufs[s], sems[s])

        for e in (0, 1):
            for h in (0, 1):
                copy(e, h).start()

        for e in range(epw):
            for h in (0, 1):
                copy(e, h).wait()
                buf = bufs[slot(e, h)]
                mv = mask_v[e, pl.ds(h * KH, L)]
                bv = jnp.where(mv == 0, jnp.float32(-1e30), jnp.float32(0.0))
                splats = [jnp.full((L,), bv[j], dtype=jnp.float32)
                          for j in range(KH)]

                def cbody(c, carry, buf=buf, splats=splats, e=e, h=h):
                    off = c * L
                    if h == 0:
                        acc = buf[0, pl.ds(off, L)] + splats[0]
                        k0 = 1
                    else:
                        acc = out_v[e, pl.ds(off, L)]
                        k0 = 0
                    for kk in range(k0, KH):
                        acc = jnp.maximum(acc, buf[kk, pl.ds(off, L)] + splats[kk])
                    out_v[e, pl.ds(off, L)] = acc
                    return carry

                lax.fori_loop(0, D // L, cbody, 0)
                if e + 2 < epw:
                    copy(e + 2, h).start()
        pltpu.sync_copy(out_v, out_hbm.at[pl.ds(base, epw), :])

    return run(table, idx, masks)


def kernel(mention_reprs, entities, entity_masks):
    B, M, D = mention_reprs.shape
    _, E, K = entities.shape
    table = mention_reprs.reshape(B * M, D)
    idx = (entities + (jnp.arange(B, dtype=jnp.int32) * M)[:, None, None]
           ).reshape(B * E, K)
    masks = entity_masks.reshape(B * E, K)
    out = _entity_pool_sc(table, idx, masks)
    return out.reshape(B, E, D)
